# baseline (speedup 1.0000x reference)
import functools
import jax
import jax.numpy as jnp
from jax.experimental import pallas as pl
from jax.experimental.pallas import tpu as pltpu

B, T, D = 64, 32, 2048
NRE = 7
DFF = 2048
BF = 256
KF = DFF // BF
GS = 16
MG = GS * T
NG = 10


def _moe_body(ge_ref, gact_ref, grow_ref, gval_ref, gw_ref,
              x_ref, w1_ref, b1_ref, w2_ref, b2_ref,
              out_ref, xg_ref, acc_ref):
    g = pl.program_id(0)
    kf = pl.program_id(1)

    @pl.when(gact_ref[g] > 0)
    def _():
        @pl.when(kf == 0)
        def _():
            for s in range(GS):
                xg_ref[s * T:(s + 1) * T] = x_ref[grow_ref[g, s] // 2]

        h = jnp.dot(xg_ref[...], w1_ref[0], preferred_element_type=jnp.float32)
        h = h + b1_ref[0, 0][None, :]
        gl = jax.nn.gelu(h)
        wcol = jnp.concatenate(
            [jnp.full((T, 1), gw_ref[g, s], jnp.float32) for s in range(GS)],
            axis=0)
        gl = gl * wcol
        contrib = jnp.dot(gl, w2_ref[0], preferred_element_type=jnp.float32)

        @pl.when(kf == 0)
        def _():
            acc_ref[...] = contrib

        @pl.when(kf > 0)
        def _():
            acc_ref[...] = acc_ref[...] + contrib

        @pl.when(kf == KF - 1)
        def _():
            total = acc_ref[...] + wcol * b2_ref[0, 0][None, :]
            for s in range(GS):
                @pl.when(gval_ref[g, s] > 0)
                def _():
                    out_ref[grow_ref[g, s]] = total[s * T:(s + 1) * T]


def _widx(kf, gact, g):
    return jnp.where(gact[g] > 0, kf, 0)


def _moe_ffn(ge, gact, grow, gval, gw, x, W1, b1r, W2, b2r):
    grid_spec = pltpu.PrefetchScalarGridSpec(
        num_scalar_prefetch=5,
        grid=(NG, KF),
        in_specs=[
            pl.BlockSpec((B, T, D),
                         lambda g, kf, ge, ga, gr, gv, gw: (0, 0, 0)),
            pl.BlockSpec((1, D, BF),
                         lambda g, kf, ge, ga, gr, gv, gw:
                         (ge[g], 0, _widx(kf, ga, g))),
            pl.BlockSpec((1, 1, BF),
                         lambda g, kf, ge, ga, gr, gv, gw:
                         (ge[g], 0, _widx(kf, ga, g))),
            pl.BlockSpec((1, BF, D),
                         lambda g, kf, ge, ga, gr, gv, gw:
                         (ge[g], _widx(kf, ga, g), 0)),
            pl.BlockSpec((1, 1, D),
                         lambda g, kf, ge, ga, gr, gv, gw: (ge[g], 0, 0)),
        ],
        out_specs=pl.BlockSpec((B, T, D),
                               lambda g, kf, ge, ga, gr, gv, gw: (0, 0, 0)),
        scratch_shapes=[
            pltpu.VMEM((MG, D), jnp.float32),
            pltpu.VMEM((MG, D), jnp.float32),
        ],
    )
    return pl.pallas_call(
        _moe_body,
        grid_spec=grid_spec,
        out_shape=jax.ShapeDtypeStruct((B, T, D), jnp.float32),
        compiler_params=pltpu.CompilerParams(
            dimension_semantics=("arbitrary", "arbitrary"),
            vmem_limit_bytes=62 * 1024 * 1024,
        ),
    )(ge, gact, grow, gval, gw, x, W1, b1r, W2, b2r)


@jax.jit
def kernel(x, Wg, W1, b1, W2, b2):
    x_avg = jnp.mean(x, axis=1)
    logits = x_avg @ Wg.T
    prob = jax.nn.softmax(logits, axis=-1)
    imp = jnp.sum(prob, axis=0)
    importance_loss = (jnp.std(imp, ddof=1) / jnp.mean(imp)) ** 2
    topv = jnp.max(prob, axis=-1)
    eid = jnp.argmax(prob, axis=-1).astype(jnp.int32)

    perm = jnp.argsort(eid, stable=True).astype(jnp.int32)
    counts = jnp.sum(eid[None, :] == jnp.arange(NRE, dtype=jnp.int32)[:, None],
                     axis=1).astype(jnp.int32)
    off = jnp.concatenate([jnp.zeros(1, jnp.int32), jnp.cumsum(counts)[:-1]])
    gpe = (counts + GS - 1) // GS
    gcum = jnp.cumsum(gpe)
    total_groups = gcum[-1]
    gids = jnp.arange(NG, dtype=jnp.int32)
    ge_raw = jnp.searchsorted(gcum, gids, side='right').astype(jnp.int32)
    valid_g = gids < total_groups
    gact = valid_g.astype(jnp.int32)
    ge = jnp.where(valid_g, ge_raw, NRE - 1).astype(jnp.int32)
    gi = gids - (gcum[ge] - gpe[ge])
    p0 = off[ge] + gi * GS
    pslots = p0[:, None] + jnp.arange(GS, dtype=jnp.int32)[None, :]
    gval = (pslots < (off[ge] + counts[ge])[:, None]) & valid_g[:, None]
    pclamp = jnp.minimum(pslots, B - 1)
    grow = perm[pclamp]
    gw = prob[grow // 2, ge[:, None]]
    gval = gval.astype(jnp.int32)

    b1r = b1.reshape(NRE, 1, DFF)
    b2r = b2.reshape(NRE, 1, D)
    output = _moe_ffn(ge, gact, grow, gval, gw, x, W1, b1r, W2, b2r)

    beam_scores = topv
    expert_route = eid[:, None]
    beam_idx = jnp.arange(B, dtype=jnp.int32)
    return (output, beam_scores, expert_route, beam_idx, importance_loss)


# speedup vs baseline: 1.4901x; 1.4901x over previous
import functools
import jax
import jax.numpy as jnp
from jax.experimental import pallas as pl
from jax.experimental.pallas import tpu as pltpu

B, T, D = 64, 32, 2048
NRE = 7
DFF = 2048
BF = 512
KF = DFF // BF
GS = 16
MG = GS * T
NG = 10


def _moe_body(ge_ref, gact_ref, grow_ref, gval_ref, gw_ref,
              x_ref, w1_ref, b1_ref, w2_ref, b2_ref,
              out_ref, xg_ref, acc_ref):
    g = pl.program_id(0)
    kf = pl.program_id(1)

    @pl.when(gact_ref[g] > 0)
    def _():
        @pl.when(kf == 0)
        def _():
            for s in range(GS):
                xg_ref[s * T:(s + 1) * T] = x_ref[grow_ref[g, s] // 2]

        h = jnp.dot(xg_ref[...], w1_ref[0], preferred_element_type=jnp.float32)
        h = h + b1_ref[0, 0][None, :]
        gl = jax.nn.gelu(h)
        wcol = jnp.concatenate(
            [jnp.full((T, 1), gw_ref[g, s], jnp.float32) for s in range(GS)],
            axis=0)
        gl = gl * wcol
        contrib = jnp.dot(gl, w2_ref[0], preferred_element_type=jnp.float32)

        @pl.when(kf == 0)
        def _():
            acc_ref[...] = contrib

        @pl.when(kf > 0)
        def _():
            acc_ref[...] = acc_ref[...] + contrib

        @pl.when(kf == KF - 1)
        def _():
            total = acc_ref[...] + wcol * b2_ref[0, 0][None, :]
            for s in range(GS):
                @pl.when(gval_ref[g, s] > 0)
                def _():
                    out_ref[grow_ref[g, s]] = total[s * T:(s + 1) * T]


def _widx(kf, gact, g):
    return jnp.where(gact[g] > 0, kf, 0)


def _moe_ffn(ge, gact, grow, gval, gw, x, W1, b1r, W2, b2r):
    grid_spec = pltpu.PrefetchScalarGridSpec(
        num_scalar_prefetch=5,
        grid=(NG, KF),
        in_specs=[
            pl.BlockSpec((B, T, D),
                         lambda g, kf, ge, ga, gr, gv, gw: (0, 0, 0)),
            pl.BlockSpec((1, D, BF),
                         lambda g, kf, ge, ga, gr, gv, gw:
                         (ge[g], 0, _widx(kf, ga, g))),
            pl.BlockSpec((1, 1, BF),
                         lambda g, kf, ge, ga, gr, gv, gw:
                         (ge[g], 0, _widx(kf, ga, g))),
            pl.BlockSpec((1, BF, D),
                         lambda g, kf, ge, ga, gr, gv, gw:
                         (ge[g], _widx(kf, ga, g), 0)),
            pl.BlockSpec((1, 1, D),
                         lambda g, kf, ge, ga, gr, gv, gw: (ge[g], 0, 0)),
        ],
        out_specs=pl.BlockSpec((B, T, D),
                               lambda g, kf, ge, ga, gr, gv, gw: (0, 0, 0)),
        scratch_shapes=[
            pltpu.VMEM((MG, D), jnp.float32),
            pltpu.VMEM((MG, D), jnp.float32),
        ],
    )
    return pl.pallas_call(
        _moe_body,
        grid_spec=grid_spec,
        out_shape=jax.ShapeDtypeStruct((B, T, D), jnp.float32),
        compiler_params=pltpu.CompilerParams(
            dimension_semantics=("arbitrary", "arbitrary"),
            vmem_limit_bytes=67000000,
        ),
    )(ge, gact, grow, gval, gw, x, W1, b1r, W2, b2r)


@jax.jit
def kernel(x, Wg, W1, b1, W2, b2):
    x_avg = jnp.mean(x, axis=1)
    logits = x_avg @ Wg.T
    prob = jax.nn.softmax(logits, axis=-1)
    imp = jnp.sum(prob, axis=0)
    importance_loss = (jnp.std(imp, ddof=1) / jnp.mean(imp)) ** 2
    topv = jnp.max(prob, axis=-1)
    eid = jnp.argmax(prob, axis=-1).astype(jnp.int32)

    perm = jnp.argsort(eid, stable=True).astype(jnp.int32)
    counts = jnp.sum(eid[None, :] == jnp.arange(NRE, dtype=jnp.int32)[:, None],
                     axis=1).astype(jnp.int32)
    off = jnp.concatenate([jnp.zeros(1, jnp.int32), jnp.cumsum(counts)[:-1]])
    gpe = (counts + GS - 1) // GS
    gcum = jnp.cumsum(gpe)
    total_groups = gcum[-1]
    gids = jnp.arange(NG, dtype=jnp.int32)
    ge_raw = jnp.searchsorted(gcum, gids, side='right').astype(jnp.int32)
    valid_g = gids < total_groups
    gact = valid_g.astype(jnp.int32)
    ge = jnp.where(valid_g, ge_raw, NRE - 1).astype(jnp.int32)
    gi = gids - (gcum[ge] - gpe[ge])
    p0 = off[ge] + gi * GS
    pslots = p0[:, None] + jnp.arange(GS, dtype=jnp.int32)[None, :]
    gval = (pslots < (off[ge] + counts[ge])[:, None]) & valid_g[:, None]
    pclamp = jnp.minimum(pslots, B - 1)
    grow = perm[pclamp]
    gw = prob[grow // 2, ge[:, None]]
    gval = gval.astype(jnp.int32)

    b1r = b1.reshape(NRE, 1, DFF)
    b2r = b2.reshape(NRE, 1, D)
    output = _moe_ffn(ge, gact, grow, gval, gw, x, W1, b1r, W2, b2r)

    beam_scores = topv
    expert_route = eid[:, None]
    beam_idx = jnp.arange(B, dtype=jnp.int32)
    return (output, beam_scores, expert_route, beam_idx, importance_loss)
